# baseline (speedup 1.0000x reference)
import jax
import jax.numpy as jnp
from jax import lax
from jax.experimental import pallas as pl
from jax.experimental.pallas import tpu as pltpu
from jax.experimental.pallas import tpu_sc as plsc

NE = 320000
NN = 10000
D = 128

NC = 2
NS = 16
NW = NC * NS
EPW = NE // NW
GS = 80
ISROWS = EPW // GS
NSCH = EPW // GS

RPS_A = 632
RPS_B = NN - 15 * RPS_A

_mesh = plsc.VectorSubcoreMesh(core_axis_name="c", subcore_axis_name="s")


def _scatter_body(eh_hbm, dst_hbm, zero_hbm, parts_hbm,
                  idx_v, ebuf_a, ebuf_b, sem_a, sem_b, sem_sa, sem_sb,
                  mail_sh):
    cid = lax.axis_index("c")
    sid = lax.axis_index("s")
    wid = cid * NS + sid

    @pl.when(sid < NS - 1)
    def _():
        pltpu.sync_copy(zero_hbm.at[pl.ds(sid * RPS_A, RPS_A)],
                        mail_sh.at[pl.ds(sid * RPS_A, RPS_A)])

    @pl.when(sid == NS - 1)
    def _():
        pltpu.sync_copy(zero_hbm.at[pl.ds(15 * RPS_A, RPS_B)],
                        mail_sh.at[pl.ds(15 * RPS_A, RPS_B)])

    plsc.subcore_barrier()

    pltpu.sync_copy(dst_hbm.at[wid], idx_v)
    ebase = wid * EPW

    def load(c, buf, sem):
        pltpu.async_copy(eh_hbm.at[pl.ds(ebase + c * GS, GS)], buf, sem)

    def wait_load(c, buf, sem):
        pltpu.make_async_copy(eh_hbm.at[pl.ds(ebase + c * GS, GS)],
                              buf, sem).wait()

    def scatter(c, buf, sem):
        pltpu.async_copy(buf, mail_sh.at[idx_v.at[c]], sem, add=True)

    def wait_scatter(c, buf, sem):
        pltpu.make_async_copy(buf, mail_sh.at[idx_v.at[c]], sem).wait()

    load(0, ebuf_a, sem_a)
    load(1, ebuf_b, sem_b)

    def pair(i, carry):
        c = 2 * i
        wait_load(c, ebuf_a, sem_a)
        scatter(c, ebuf_a, sem_sa)
        wait_load(c + 1, ebuf_b, sem_b)
        scatter(c + 1, ebuf_b, sem_sb)
        wait_scatter(c, ebuf_a, sem_sa)
        load(c + 2, ebuf_a, sem_a)
        wait_scatter(c + 1, ebuf_b, sem_sb)
        load(c + 3, ebuf_b, sem_b)
        return carry

    lax.fori_loop(0, (NSCH - 3) // 2, pair, 0)

    c = NSCH - 3
    wait_load(c, ebuf_a, sem_a)
    scatter(c, ebuf_a, sem_sa)
    wait_load(c + 1, ebuf_b, sem_b)
    scatter(c + 1, ebuf_b, sem_sb)
    wait_scatter(c, ebuf_a, sem_sa)
    load(c + 2, ebuf_a, sem_a)
    wait_scatter(c + 1, ebuf_b, sem_sb)
    wait_load(c + 2, ebuf_a, sem_a)
    scatter(c + 2, ebuf_a, sem_sa)
    wait_scatter(c + 2, ebuf_a, sem_sa)

    plsc.subcore_barrier()

    @pl.when(sid < NS - 1)
    def _():
        pltpu.sync_copy(mail_sh.at[pl.ds(sid * RPS_A, RPS_A)],
                        parts_hbm.at[cid, pl.ds(sid * RPS_A, RPS_A)])

    @pl.when(sid == NS - 1)
    def _():
        pltpu.sync_copy(mail_sh.at[pl.ds(15 * RPS_A, RPS_B)],
                        parts_hbm.at[cid, pl.ds(15 * RPS_A, RPS_B)])


_scatter = pl.kernel(
    _scatter_body,
    out_type=jax.ShapeDtypeStruct((NC, NN, D), jnp.float32),
    mesh=_mesh,
    scratch_types=[
        pltpu.VMEM((ISROWS, GS), jnp.int32),
        pltpu.VMEM((GS, D), jnp.float32),
        pltpu.VMEM((GS, D), jnp.float32),
        pltpu.SemaphoreType.DMA,
        pltpu.SemaphoreType.DMA,
        pltpu.SemaphoreType.DMA,
        pltpu.SemaphoreType.DMA,
        pltpu.VMEM_SHARED((NN, D), jnp.float32),
    ],
)


def _gather_body(mw_hbm, src_hbm, out_hbm,
                 idx_v, gbuf_a, gbuf_b, sem_a, sem_b, mw_sh):
    cid = lax.axis_index("c")
    sid = lax.axis_index("s")
    wid = cid * NS + sid

    @pl.when(sid < NS - 1)
    def _():
        pltpu.sync_copy(mw_hbm.at[pl.ds(sid * RPS_A, RPS_A)],
                        mw_sh.at[pl.ds(sid * RPS_A, RPS_A)])

    @pl.when(sid == NS - 1)
    def _():
        pltpu.sync_copy(mw_hbm.at[pl.ds(15 * RPS_A, RPS_B)],
                        mw_sh.at[pl.ds(15 * RPS_A, RPS_B)])

    ebase = wid * EPW
    pltpu.sync_copy(src_hbm.at[pl.ds(ebase, EPW)], idx_v)
    plsc.subcore_barrier()

    def gath(c, buf, sem):
        pltpu.async_copy(mw_sh.at[idx_v.at[pl.ds(c * GS, GS)]], buf, sem)

    def wait_gath(c, buf, sem):
        pltpu.make_async_copy(mw_sh.at[idx_v.at[pl.ds(c * GS, GS)]],
                              buf, sem).wait()

    def store(c, buf):
        pltpu.sync_copy(buf, out_hbm.at[pl.ds(ebase + c * GS, GS)])

    gath(0, gbuf_a, sem_a)

    def pair(i, carry):
        c = 2 * i
        gath(c + 1, gbuf_b, sem_b)
        wait_gath(c, gbuf_a, sem_a)
        store(c, gbuf_a)
        gath(c + 2, gbuf_a, sem_a)
        wait_gath(c + 1, gbuf_b, sem_b)
        store(c + 1, gbuf_b)
        return carry

    lax.fori_loop(0, (NSCH - 3) // 2, pair, 0)

    c = NSCH - 3
    gath(c + 1, gbuf_b, sem_b)
    wait_gath(c, gbuf_a, sem_a)
    store(c, gbuf_a)
    gath(c + 2, gbuf_a, sem_a)
    wait_gath(c + 1, gbuf_b, sem_b)
    store(c + 1, gbuf_b)
    wait_gath(c + 2, gbuf_a, sem_a)
    store(c + 2, gbuf_a)


_gather = pl.kernel(
    _gather_body,
    out_type=jax.ShapeDtypeStruct((NE, D), jnp.float32),
    mesh=_mesh,
    scratch_types=[
        pltpu.VMEM((EPW,), jnp.int32),
        pltpu.VMEM((GS, D), jnp.float32),
        pltpu.VMEM((GS, D), jnp.float32),
        pltpu.SemaphoreType.DMA,
        pltpu.SemaphoreType.DMA,
        pltpu.VMEM_SHARED((NN, D), jnp.float32),
    ],
)


def _mailw_body(parts_ref, w_ref, o_ref):
    p = parts_ref[0] + parts_ref[1]
    o_ref[...] = jnp.dot(p, w_ref[...], preferred_element_type=jnp.float32)


def _fused_body(g_ref, eh_ref, ei_ref, w_ref, o_ref):
    o_ref[...] = (g_ref[...]
                  - jnp.dot(eh_ref[...], w_ref[...],
                            preferred_element_type=jnp.float32)
                  + ei_ref[...])


_MAILW_BLK = 1000
_FUSE_BLK = 8000


def kernel(edge_hidden, edge_init, W, edge_index):
    src = edge_index[0]
    dst = edge_index[1].reshape(NW, ISROWS, GS)
    zeros = jnp.zeros((NN, D), jnp.float32)

    parts = _scatter(edge_hidden, dst, zeros)

    mail_w = pl.pallas_call(
        _mailw_body,
        grid=(NN // _MAILW_BLK,),
        in_specs=[
            pl.BlockSpec((NC, _MAILW_BLK, D), lambda i: (0, i, 0)),
            pl.BlockSpec((D, D), lambda i: (0, 0)),
        ],
        out_specs=pl.BlockSpec((_MAILW_BLK, D), lambda i: (i, 0)),
        out_shape=jax.ShapeDtypeStruct((NN, D), jnp.float32),
    )(parts, W)

    gathered = _gather(mail_w, src)

    out = pl.pallas_call(
        _fused_body,
        grid=(NE // _FUSE_BLK,),
        in_specs=[
            pl.BlockSpec((_FUSE_BLK, D), lambda i: (i, 0)),
            pl.BlockSpec((_FUSE_BLK, D), lambda i: (i, 0)),
            pl.BlockSpec((_FUSE_BLK, D), lambda i: (i, 0)),
            pl.BlockSpec((D, D), lambda i: (0, 0)),
        ],
        out_specs=pl.BlockSpec((_FUSE_BLK, D), lambda i: (i, 0)),
        out_shape=jax.ShapeDtypeStruct((NE, D), jnp.float32),
    )(gathered, edge_hidden, edge_init, W)

    return out


# speedup vs baseline: 1.0868x; 1.0868x over previous
import jax
import jax.numpy as jnp
from jax import lax
from jax.experimental import pallas as pl
from jax.experimental.pallas import tpu as pltpu
from jax.experimental.pallas import tpu_sc as plsc

NE = 320000
NN = 10000
D = 128

NC = 2
NS = 16
NW = NC * NS
EPW = NE // NW
GS = 80
ISROWS = EPW // GS
NSCH = EPW // GS

RPS_A = 632
RPS_B = NN - 15 * RPS_A

_mesh = plsc.VectorSubcoreMesh(core_axis_name="c", subcore_axis_name="s")


def _scatter_body(eh_hbm, dst_hbm, zero_hbm, parts_hbm,
                  idx_v, ebuf_a, ebuf_b, sem_a, sem_b, mail_sh):
    cid = lax.axis_index("c")
    sid = lax.axis_index("s")
    wid = cid * NS + sid

    @pl.when(sid < NS - 1)
    def _():
        pltpu.sync_copy(zero_hbm.at[pl.ds(sid * RPS_A, RPS_A)],
                        mail_sh.at[pl.ds(sid * RPS_A, RPS_A)])

    @pl.when(sid == NS - 1)
    def _():
        pltpu.sync_copy(zero_hbm.at[pl.ds(15 * RPS_A, RPS_B)],
                        mail_sh.at[pl.ds(15 * RPS_A, RPS_B)])

    plsc.subcore_barrier()

    pltpu.sync_copy(dst_hbm.at[wid], idx_v)
    ebase = wid * EPW

    def load(c, buf, sem):
        pltpu.async_copy(eh_hbm.at[pl.ds(ebase + c * GS, GS)], buf, sem)

    def wait_load(c, buf, sem):
        pltpu.make_async_copy(eh_hbm.at[pl.ds(ebase + c * GS, GS)],
                              buf, sem).wait()

    def scatter(c, buf):
        pltpu.sync_copy(buf, mail_sh.at[idx_v.at[c]], add=True)

    load(0, ebuf_a, sem_a)

    def pair(i, carry):
        c = 2 * i
        load(c + 1, ebuf_b, sem_b)
        wait_load(c, ebuf_a, sem_a)
        scatter(c, ebuf_a)
        load(c + 2, ebuf_a, sem_a)
        wait_load(c + 1, ebuf_b, sem_b)
        scatter(c + 1, ebuf_b)
        return carry

    lax.fori_loop(0, (NSCH - 3) // 2, pair, 0)

    c = NSCH - 3
    load(c + 1, ebuf_b, sem_b)
    wait_load(c, ebuf_a, sem_a)
    scatter(c, ebuf_a)
    load(c + 2, ebuf_a, sem_a)
    wait_load(c + 1, ebuf_b, sem_b)
    scatter(c + 1, ebuf_b)
    wait_load(c + 2, ebuf_a, sem_a)
    scatter(c + 2, ebuf_a)

    plsc.subcore_barrier()

    @pl.when(sid < NS - 1)
    def _():
        pltpu.sync_copy(mail_sh.at[pl.ds(sid * RPS_A, RPS_A)],
                        parts_hbm.at[cid, pl.ds(sid * RPS_A, RPS_A)])

    @pl.when(sid == NS - 1)
    def _():
        pltpu.sync_copy(mail_sh.at[pl.ds(15 * RPS_A, RPS_B)],
                        parts_hbm.at[cid, pl.ds(15 * RPS_A, RPS_B)])


_scatter = pl.kernel(
    _scatter_body,
    out_type=jax.ShapeDtypeStruct((NC, NN, D), jnp.float32),
    mesh=_mesh,
    scratch_types=[
        pltpu.VMEM((ISROWS, GS), jnp.int32),
        pltpu.VMEM((GS, D), jnp.float32),
        pltpu.VMEM((GS, D), jnp.float32),
        pltpu.SemaphoreType.DMA,
        pltpu.SemaphoreType.DMA,
        pltpu.VMEM_SHARED((NN, D), jnp.float32),
    ],
)


def _gather_body(mw_hbm, src_hbm, out_hbm,
                 idx_v, gbuf_a, gbuf_b, sem_a, sem_b, mw_sh):
    cid = lax.axis_index("c")
    sid = lax.axis_index("s")
    wid = cid * NS + sid

    @pl.when(sid < NS - 1)
    def _():
        pltpu.sync_copy(mw_hbm.at[pl.ds(sid * RPS_A, RPS_A)],
                        mw_sh.at[pl.ds(sid * RPS_A, RPS_A)])

    @pl.when(sid == NS - 1)
    def _():
        pltpu.sync_copy(mw_hbm.at[pl.ds(15 * RPS_A, RPS_B)],
                        mw_sh.at[pl.ds(15 * RPS_A, RPS_B)])

    ebase = wid * EPW
    pltpu.sync_copy(src_hbm.at[pl.ds(ebase, EPW)], idx_v)
    plsc.subcore_barrier()

    def gath(c, buf, sem):
        pltpu.async_copy(mw_sh.at[idx_v.at[pl.ds(c * GS, GS)]], buf, sem)

    def wait_gath(c, buf, sem):
        pltpu.make_async_copy(mw_sh.at[idx_v.at[pl.ds(c * GS, GS)]],
                              buf, sem).wait()

    def store(c, buf):
        pltpu.sync_copy(buf, out_hbm.at[pl.ds(ebase + c * GS, GS)])

    gath(0, gbuf_a, sem_a)

    def pair(i, carry):
        c = 2 * i
        gath(c + 1, gbuf_b, sem_b)
        wait_gath(c, gbuf_a, sem_a)
        store(c, gbuf_a)
        gath(c + 2, gbuf_a, sem_a)
        wait_gath(c + 1, gbuf_b, sem_b)
        store(c + 1, gbuf_b)
        return carry

    lax.fori_loop(0, (NSCH - 3) // 2, pair, 0)

    c = NSCH - 3
    gath(c + 1, gbuf_b, sem_b)
    wait_gath(c, gbuf_a, sem_a)
    store(c, gbuf_a)
    gath(c + 2, gbuf_a, sem_a)
    wait_gath(c + 1, gbuf_b, sem_b)
    store(c + 1, gbuf_b)
    wait_gath(c + 2, gbuf_a, sem_a)
    store(c + 2, gbuf_a)


_gather = pl.kernel(
    _gather_body,
    out_type=jax.ShapeDtypeStruct((NE, D), jnp.float32),
    mesh=_mesh,
    scratch_types=[
        pltpu.VMEM((EPW,), jnp.int32),
        pltpu.VMEM((GS, D), jnp.float32),
        pltpu.VMEM((GS, D), jnp.float32),
        pltpu.SemaphoreType.DMA,
        pltpu.SemaphoreType.DMA,
        pltpu.VMEM_SHARED((NN, D), jnp.float32),
    ],
)


def _mailw_body(parts_ref, w_ref, o_ref):
    p = parts_ref[0] + parts_ref[1]
    o_ref[...] = jnp.dot(p, w_ref[...], preferred_element_type=jnp.float32)


def _fused_body(g_ref, eh_ref, ei_ref, w_ref, o_ref):
    o_ref[...] = (g_ref[...]
                  - jnp.dot(eh_ref[...], w_ref[...],
                            preferred_element_type=jnp.float32)
                  + ei_ref[...])


_MAILW_BLK = 1000
_FUSE_BLK = 4000


def kernel(edge_hidden, edge_init, W, edge_index):
    src = edge_index[0]
    dst = edge_index[1].reshape(NW, ISROWS, GS)
    zeros = jnp.zeros((NN, D), jnp.float32)

    parts = _scatter(edge_hidden, dst, zeros)

    mail_w = pl.pallas_call(
        _mailw_body,
        grid=(NN // _MAILW_BLK,),
        in_specs=[
            pl.BlockSpec((NC, _MAILW_BLK, D), lambda i: (0, i, 0)),
            pl.BlockSpec((D, D), lambda i: (0, 0)),
        ],
        out_specs=pl.BlockSpec((_MAILW_BLK, D), lambda i: (i, 0)),
        out_shape=jax.ShapeDtypeStruct((NN, D), jnp.float32),
    )(parts, W)

    gathered = _gather(mail_w, src)

    out = pl.pallas_call(
        _fused_body,
        grid=(NE // _FUSE_BLK,),
        in_specs=[
            pl.BlockSpec((_FUSE_BLK, D), lambda i: (i, 0)),
            pl.BlockSpec((_FUSE_BLK, D), lambda i: (i, 0)),
            pl.BlockSpec((_FUSE_BLK, D), lambda i: (i, 0)),
            pl.BlockSpec((D, D), lambda i: (0, 0)),
        ],
        out_specs=pl.BlockSpec((_FUSE_BLK, D), lambda i: (i, 0)),
        out_shape=jax.ShapeDtypeStruct((NE, D), jnp.float32),
    )(gathered, edge_hidden, edge_init, W)

    return out


# speedup vs baseline: 1.0909x; 1.0038x over previous
import jax
import jax.numpy as jnp
from jax import lax
from jax.experimental import pallas as pl
from jax.experimental.pallas import tpu as pltpu
from jax.experimental.pallas import tpu_sc as plsc

NE = 320000
NN = 10000
D = 128

NC = 2
NS = 16
NW = NC * NS
EPW = NE // NW
GS = 80
ISROWS = EPW // GS
NSCH = EPW // GS

RPS_A = 632
RPS_B = NN - 15 * RPS_A

_mesh = plsc.VectorSubcoreMesh(core_axis_name="c", subcore_axis_name="s")


def _scatter_body(eh_hbm, dst_hbm, zero_hbm, parts_hbm,
                  idx_v, ebuf_a, ebuf_b, sem_a, sem_b, mail_sh):
    cid = lax.axis_index("c")
    sid = lax.axis_index("s")
    wid = cid * NS + sid

    @pl.when(sid < NS - 1)
    def _():
        pltpu.sync_copy(zero_hbm.at[pl.ds(sid * RPS_A, RPS_A)],
                        mail_sh.at[pl.ds(sid * RPS_A, RPS_A)])

    @pl.when(sid == NS - 1)
    def _():
        pltpu.sync_copy(zero_hbm.at[pl.ds(15 * RPS_A, RPS_B)],
                        mail_sh.at[pl.ds(15 * RPS_A, RPS_B)])

    plsc.subcore_barrier()

    pltpu.sync_copy(dst_hbm.at[wid], idx_v)
    ebase = wid * EPW

    def load(c, buf, sem):
        pltpu.async_copy(eh_hbm.at[pl.ds(ebase + c * GS, GS)], buf, sem)

    def wait_load(c, buf, sem):
        pltpu.make_async_copy(eh_hbm.at[pl.ds(ebase + c * GS, GS)],
                              buf, sem).wait()

    def scatter(c, buf):
        pltpu.sync_copy(buf, mail_sh.at[idx_v.at[c]], add=True)

    load(0, ebuf_a, sem_a)

    def pair(i, carry):
        c = 2 * i
        load(c + 1, ebuf_b, sem_b)
        wait_load(c, ebuf_a, sem_a)
        scatter(c, ebuf_a)
        load(c + 2, ebuf_a, sem_a)
        wait_load(c + 1, ebuf_b, sem_b)
        scatter(c + 1, ebuf_b)
        return carry

    lax.fori_loop(0, (NSCH - 3) // 2, pair, 0)

    c = NSCH - 3
    load(c + 1, ebuf_b, sem_b)
    wait_load(c, ebuf_a, sem_a)
    scatter(c, ebuf_a)
    load(c + 2, ebuf_a, sem_a)
    wait_load(c + 1, ebuf_b, sem_b)
    scatter(c + 1, ebuf_b)
    wait_load(c + 2, ebuf_a, sem_a)
    scatter(c + 2, ebuf_a)

    plsc.subcore_barrier()

    @pl.when(sid < NS - 1)
    def _():
        pltpu.sync_copy(mail_sh.at[pl.ds(sid * RPS_A, RPS_A)],
                        parts_hbm.at[cid, pl.ds(sid * RPS_A, RPS_A)])

    @pl.when(sid == NS - 1)
    def _():
        pltpu.sync_copy(mail_sh.at[pl.ds(15 * RPS_A, RPS_B)],
                        parts_hbm.at[cid, pl.ds(15 * RPS_A, RPS_B)])


_scatter = pl.kernel(
    _scatter_body,
    out_type=jax.ShapeDtypeStruct((NC, NN, D), jnp.float32),
    mesh=_mesh,
    scratch_types=[
        pltpu.VMEM((ISROWS, GS), jnp.int32),
        pltpu.VMEM((GS, D), jnp.float32),
        pltpu.VMEM((GS, D), jnp.float32),
        pltpu.SemaphoreType.DMA,
        pltpu.SemaphoreType.DMA,
        pltpu.VMEM_SHARED((NN, D), jnp.float32),
    ],
)


def _gather_body(mw_hbm, src_hbm, out_hbm,
                 idx_v, gbuf_a, gbuf_b, sem_a, sem_b, mw_sh):
    cid = lax.axis_index("c")
    sid = lax.axis_index("s")
    wid = cid * NS + sid

    @pl.when(sid < NS - 1)
    def _():
        pltpu.sync_copy(mw_hbm.at[pl.ds(sid * RPS_A, RPS_A)],
                        mw_sh.at[pl.ds(sid * RPS_A, RPS_A)])

    @pl.when(sid == NS - 1)
    def _():
        pltpu.sync_copy(mw_hbm.at[pl.ds(15 * RPS_A, RPS_B)],
                        mw_sh.at[pl.ds(15 * RPS_A, RPS_B)])

    ebase = wid * EPW
    pltpu.sync_copy(src_hbm.at[pl.ds(ebase, EPW)], idx_v)
    plsc.subcore_barrier()

    def gath(c, buf, sem):
        pltpu.async_copy(mw_sh.at[idx_v.at[pl.ds(c * GS, GS)]], buf, sem)

    def wait_gath(c, buf, sem):
        pltpu.make_async_copy(mw_sh.at[idx_v.at[pl.ds(c * GS, GS)]],
                              buf, sem).wait()

    def store(c, buf):
        pltpu.sync_copy(buf, out_hbm.at[pl.ds(ebase + c * GS, GS)])

    gath(0, gbuf_a, sem_a)

    def pair(i, carry):
        c = 2 * i
        gath(c + 1, gbuf_b, sem_b)
        wait_gath(c, gbuf_a, sem_a)
        store(c, gbuf_a)
        gath(c + 2, gbuf_a, sem_a)
        wait_gath(c + 1, gbuf_b, sem_b)
        store(c + 1, gbuf_b)
        return carry

    lax.fori_loop(0, (NSCH - 3) // 2, pair, 0)

    c = NSCH - 3
    gath(c + 1, gbuf_b, sem_b)
    wait_gath(c, gbuf_a, sem_a)
    store(c, gbuf_a)
    gath(c + 2, gbuf_a, sem_a)
    wait_gath(c + 1, gbuf_b, sem_b)
    store(c + 1, gbuf_b)
    wait_gath(c + 2, gbuf_a, sem_a)
    store(c + 2, gbuf_a)


_gather = pl.kernel(
    _gather_body,
    out_type=jax.ShapeDtypeStruct((NE, D), jnp.float32),
    mesh=_mesh,
    scratch_types=[
        pltpu.VMEM((EPW,), jnp.int32),
        pltpu.VMEM((GS, D), jnp.float32),
        pltpu.VMEM((GS, D), jnp.float32),
        pltpu.SemaphoreType.DMA,
        pltpu.SemaphoreType.DMA,
        pltpu.VMEM_SHARED((NN, D), jnp.float32),
    ],
)


def _mailw_body(parts_ref, w_ref, o_ref):
    p = parts_ref[0] + parts_ref[1]
    o_ref[...] = jnp.dot(p, w_ref[...], preferred_element_type=jnp.float32)


def _fused_body(g_ref, eh_ref, ei_ref, w_ref, o_ref):
    o_ref[...] = (g_ref[...]
                  - jnp.dot(eh_ref[...], w_ref[...],
                            preferred_element_type=jnp.float32)
                  + ei_ref[...])


_MAILW_BLK = 1000
_FUSE_BLK = 6400


def kernel(edge_hidden, edge_init, W, edge_index):
    src = edge_index[0]
    dst = edge_index[1].reshape(NW, ISROWS, GS)
    zeros = jnp.zeros((NN, D), jnp.float32)

    parts = _scatter(edge_hidden, dst, zeros)

    mail_w = pl.pallas_call(
        _mailw_body,
        grid=(NN // _MAILW_BLK,),
        in_specs=[
            pl.BlockSpec((NC, _MAILW_BLK, D), lambda i: (0, i, 0)),
            pl.BlockSpec((D, D), lambda i: (0, 0)),
        ],
        out_specs=pl.BlockSpec((_MAILW_BLK, D), lambda i: (i, 0)),
        out_shape=jax.ShapeDtypeStruct((NN, D), jnp.float32),
    )(parts, W)

    gathered = _gather(mail_w, src)

    out = pl.pallas_call(
        _fused_body,
        grid=(NE // _FUSE_BLK,),
        in_specs=[
            pl.BlockSpec((_FUSE_BLK, D), lambda i: (i, 0)),
            pl.BlockSpec((_FUSE_BLK, D), lambda i: (i, 0)),
            pl.BlockSpec((_FUSE_BLK, D), lambda i: (i, 0)),
            pl.BlockSpec((D, D), lambda i: (0, 0)),
        ],
        out_specs=pl.BlockSpec((_FUSE_BLK, D), lambda i: (i, 0)),
        out_shape=jax.ShapeDtypeStruct((NE, D), jnp.float32),
    )(gathered, edge_hidden, edge_init, W)

    return out


# speedup vs baseline: 1.1471x; 1.0515x over previous
import jax
import jax.numpy as jnp
from jax import lax
from jax.experimental import pallas as pl
from jax.experimental.pallas import tpu as pltpu
from jax.experimental.pallas import tpu_sc as plsc

NE = 320000
NN = 10000
D = 128

NC = 2
NS = 16
NW = NC * NS
EPW = NE // NW
GS = 80
ISROWS = EPW // GS
NSCH = EPW // GS

RPS_A = 632
RPS_B = NN - 15 * RPS_A

_mesh = plsc.VectorSubcoreMesh(core_axis_name="c", subcore_axis_name="s")


def _scatter_body(eh_hbm, dst_hbm, zero_hbm, parts_hbm,
                  idx_v, ebuf_a, ebuf_b, ebuf_c, sem_a, sem_b, sem_c,
                  mail_sh):
    cid = lax.axis_index("c")
    sid = lax.axis_index("s")
    wid = cid * NS + sid

    @pl.when(sid < NS - 1)
    def _():
        pltpu.sync_copy(zero_hbm.at[pl.ds(sid * RPS_A, RPS_A)],
                        mail_sh.at[pl.ds(sid * RPS_A, RPS_A)])

    @pl.when(sid == NS - 1)
    def _():
        pltpu.sync_copy(zero_hbm.at[pl.ds(15 * RPS_A, RPS_B)],
                        mail_sh.at[pl.ds(15 * RPS_A, RPS_B)])

    plsc.subcore_barrier()

    pltpu.sync_copy(dst_hbm.at[wid], idx_v)
    ebase = wid * EPW

    def load(c, buf, sem):
        pltpu.async_copy(eh_hbm.at[pl.ds(ebase + c * GS, GS)], buf, sem)

    def wait_load(c, buf, sem):
        pltpu.make_async_copy(eh_hbm.at[pl.ds(ebase + c * GS, GS)],
                              buf, sem).wait()

    def scatter(c, buf):
        pltpu.sync_copy(buf, mail_sh.at[idx_v.at[c]], add=True)

    load(0, ebuf_a, sem_a)
    load(1, ebuf_b, sem_b)

    def triple(i, carry):
        c = 3 * i
        load(c + 2, ebuf_c, sem_c)
        wait_load(c, ebuf_a, sem_a)
        scatter(c, ebuf_a)
        load(c + 3, ebuf_a, sem_a)
        wait_load(c + 1, ebuf_b, sem_b)
        scatter(c + 1, ebuf_b)
        load(c + 4, ebuf_b, sem_b)
        wait_load(c + 2, ebuf_c, sem_c)
        scatter(c + 2, ebuf_c)
        return carry

    lax.fori_loop(0, (NSCH - 2) // 3, triple, 0)

    c = NSCH - 2
    wait_load(c, ebuf_a, sem_a)
    scatter(c, ebuf_a)
    wait_load(c + 1, ebuf_b, sem_b)
    scatter(c + 1, ebuf_b)

    plsc.subcore_barrier()

    @pl.when(sid < NS - 1)
    def _():
        pltpu.sync_copy(mail_sh.at[pl.ds(sid * RPS_A, RPS_A)],
                        parts_hbm.at[cid, pl.ds(sid * RPS_A, RPS_A)])

    @pl.when(sid == NS - 1)
    def _():
        pltpu.sync_copy(mail_sh.at[pl.ds(15 * RPS_A, RPS_B)],
                        parts_hbm.at[cid, pl.ds(15 * RPS_A, RPS_B)])


_scatter = pl.kernel(
    _scatter_body,
    out_type=jax.ShapeDtypeStruct((NC, NN, D), jnp.float32),
    mesh=_mesh,
    scratch_types=[
        pltpu.VMEM((ISROWS, GS), jnp.int32),
        pltpu.VMEM((GS, D), jnp.float32),
        pltpu.VMEM((GS, D), jnp.float32),
        pltpu.VMEM((GS, D), jnp.float32),
        pltpu.SemaphoreType.DMA,
        pltpu.SemaphoreType.DMA,
        pltpu.SemaphoreType.DMA,
        pltpu.VMEM_SHARED((NN, D), jnp.float32),
    ],
)


def _gather_body(mw_hbm, src_hbm, out_hbm,
                 idx_v, gbuf_a, gbuf_b, sem_a, sem_b, mw_sh):
    cid = lax.axis_index("c")
    sid = lax.axis_index("s")
    wid = cid * NS + sid

    @pl.when(sid < NS - 1)
    def _():
        pltpu.sync_copy(mw_hbm.at[pl.ds(sid * RPS_A, RPS_A)],
                        mw_sh.at[pl.ds(sid * RPS_A, RPS_A)])

    @pl.when(sid == NS - 1)
    def _():
        pltpu.sync_copy(mw_hbm.at[pl.ds(15 * RPS_A, RPS_B)],
                        mw_sh.at[pl.ds(15 * RPS_A, RPS_B)])

    ebase = wid * EPW
    pltpu.sync_copy(src_hbm.at[pl.ds(ebase, EPW)], idx_v)
    plsc.subcore_barrier()

    def gath(c, buf, sem):
        pltpu.async_copy(mw_sh.at[idx_v.at[pl.ds(c * GS, GS)]], buf, sem)

    def wait_gath(c, buf, sem):
        pltpu.make_async_copy(mw_sh.at[idx_v.at[pl.ds(c * GS, GS)]],
                              buf, sem).wait()

    def store(c, buf):
        pltpu.sync_copy(buf, out_hbm.at[pl.ds(ebase + c * GS, GS)])

    gath(0, gbuf_a, sem_a)

    def pair(i, carry):
        c = 2 * i
        gath(c + 1, gbuf_b, sem_b)
        wait_gath(c, gbuf_a, sem_a)
        store(c, gbuf_a)
        gath(c + 2, gbuf_a, sem_a)
        wait_gath(c + 1, gbuf_b, sem_b)
        store(c + 1, gbuf_b)
        return carry

    lax.fori_loop(0, (NSCH - 3) // 2, pair, 0)

    c = NSCH - 3
    gath(c + 1, gbuf_b, sem_b)
    wait_gath(c, gbuf_a, sem_a)
    store(c, gbuf_a)
    gath(c + 2, gbuf_a, sem_a)
    wait_gath(c + 1, gbuf_b, sem_b)
    store(c + 1, gbuf_b)
    wait_gath(c + 2, gbuf_a, sem_a)
    store(c + 2, gbuf_a)


_gather = pl.kernel(
    _gather_body,
    out_type=jax.ShapeDtypeStruct((NE, D), jnp.float32),
    mesh=_mesh,
    scratch_types=[
        pltpu.VMEM((EPW,), jnp.int32),
        pltpu.VMEM((GS, D), jnp.float32),
        pltpu.VMEM((GS, D), jnp.float32),
        pltpu.SemaphoreType.DMA,
        pltpu.SemaphoreType.DMA,
        pltpu.VMEM_SHARED((NN, D), jnp.float32),
    ],
)


def _mailw_body(parts_ref, w_ref, o_ref):
    p = parts_ref[0] + parts_ref[1]
    o_ref[...] = jnp.dot(p, w_ref[...], preferred_element_type=jnp.float32)


def _fused_body(g_ref, eh_ref, ei_ref, w_ref, o_ref):
    o_ref[...] = (g_ref[...]
                  - jnp.dot(eh_ref[...], w_ref[...],
                            preferred_element_type=jnp.float32)
                  + ei_ref[...])


_MAILW_BLK = 1000
_FUSE_BLK = 6400


def kernel(edge_hidden, edge_init, W, edge_index):
    src = edge_index[0]
    dst = edge_index[1].reshape(NW, ISROWS, GS)
    zeros = jnp.zeros((NN, D), jnp.float32)

    parts = _scatter(edge_hidden, dst, zeros)

    mail_w = pl.pallas_call(
        _mailw_body,
        grid=(NN // _MAILW_BLK,),
        in_specs=[
            pl.BlockSpec((NC, _MAILW_BLK, D), lambda i: (0, i, 0)),
            pl.BlockSpec((D, D), lambda i: (0, 0)),
        ],
        out_specs=pl.BlockSpec((_MAILW_BLK, D), lambda i: (i, 0)),
        out_shape=jax.ShapeDtypeStruct((NN, D), jnp.float32),
    )(parts, W)

    gathered = _gather(mail_w, src)

    out = pl.pallas_call(
        _fused_body,
        grid=(NE // _FUSE_BLK,),
        in_specs=[
            pl.BlockSpec((_FUSE_BLK, D), lambda i: (i, 0)),
            pl.BlockSpec((_FUSE_BLK, D), lambda i: (i, 0)),
            pl.BlockSpec((_FUSE_BLK, D), lambda i: (i, 0)),
            pl.BlockSpec((D, D), lambda i: (0, 0)),
        ],
        out_specs=pl.BlockSpec((_FUSE_BLK, D), lambda i: (i, 0)),
        out_shape=jax.ShapeDtypeStruct((NE, D), jnp.float32),
    )(gathered, edge_hidden, edge_init, W)

    return out


# speedup vs baseline: 1.1600x; 1.0113x over previous
import jax
import jax.numpy as jnp
from jax import lax
from jax.experimental import pallas as pl
from jax.experimental.pallas import tpu as pltpu
from jax.experimental.pallas import tpu_sc as plsc

NE = 320000
NN = 10000
D = 128

NC = 2
NS = 16
NW = NC * NS
EPW = NE // NW
GS = 80
ISROWS = EPW // GS
NSCH = EPW // GS

RPS_A = 632
RPS_B = NN - 15 * RPS_A

_mesh = plsc.VectorSubcoreMesh(core_axis_name="c", subcore_axis_name="s")


def _scatter_body(eh_hbm, dst_hbm, zero_hbm, parts_hbm,
                  idx_v, ebuf_a, ebuf_b, ebuf_c, sem_a, sem_b, sem_c,
                  mail_sh):
    cid = lax.axis_index("c")
    sid = lax.axis_index("s")
    wid = cid * NS + sid

    @pl.when(sid < NS - 1)
    def _():
        pltpu.sync_copy(zero_hbm.at[pl.ds(sid * RPS_A, RPS_A)],
                        mail_sh.at[pl.ds(sid * RPS_A, RPS_A)])

    @pl.when(sid == NS - 1)
    def _():
        pltpu.sync_copy(zero_hbm.at[pl.ds(15 * RPS_A, RPS_B)],
                        mail_sh.at[pl.ds(15 * RPS_A, RPS_B)])

    plsc.subcore_barrier()

    pltpu.sync_copy(dst_hbm.at[wid], idx_v)
    ebase = wid * EPW

    def load(c, buf, sem):
        pltpu.async_copy(eh_hbm.at[pl.ds(ebase + c * GS, GS)], buf, sem)

    def wait_load(c, buf, sem):
        pltpu.make_async_copy(eh_hbm.at[pl.ds(ebase + c * GS, GS)],
                              buf, sem).wait()

    def scatter(c, buf):
        pltpu.sync_copy(buf, mail_sh.at[idx_v.at[c]], add=True)

    load(0, ebuf_a, sem_a)
    load(1, ebuf_b, sem_b)

    def triple(i, carry):
        c = 3 * i
        load(c + 2, ebuf_c, sem_c)
        wait_load(c, ebuf_a, sem_a)
        scatter(c, ebuf_a)
        load(c + 3, ebuf_a, sem_a)
        wait_load(c + 1, ebuf_b, sem_b)
        scatter(c + 1, ebuf_b)
        load(c + 4, ebuf_b, sem_b)
        wait_load(c + 2, ebuf_c, sem_c)
        scatter(c + 2, ebuf_c)
        return carry

    lax.fori_loop(0, (NSCH - 2) // 3, triple, 0)

    c = NSCH - 2
    wait_load(c, ebuf_a, sem_a)
    scatter(c, ebuf_a)
    wait_load(c + 1, ebuf_b, sem_b)
    scatter(c + 1, ebuf_b)

    plsc.subcore_barrier()

    @pl.when(sid < NS - 1)
    def _():
        pltpu.sync_copy(mail_sh.at[pl.ds(sid * RPS_A, RPS_A)],
                        parts_hbm.at[cid, pl.ds(sid * RPS_A, RPS_A)])

    @pl.when(sid == NS - 1)
    def _():
        pltpu.sync_copy(mail_sh.at[pl.ds(15 * RPS_A, RPS_B)],
                        parts_hbm.at[cid, pl.ds(15 * RPS_A, RPS_B)])


_scatter = pl.kernel(
    _scatter_body,
    out_type=jax.ShapeDtypeStruct((NC, NN, D), jnp.float32),
    mesh=_mesh,
    scratch_types=[
        pltpu.VMEM((ISROWS, GS), jnp.int32),
        pltpu.VMEM((GS, D), jnp.float32),
        pltpu.VMEM((GS, D), jnp.float32),
        pltpu.VMEM((GS, D), jnp.float32),
        pltpu.SemaphoreType.DMA,
        pltpu.SemaphoreType.DMA,
        pltpu.SemaphoreType.DMA,
        pltpu.VMEM_SHARED((NN, D), jnp.float32),
    ],
)


def _gather_body(mw_hbm, src_hbm, out_hbm,
                 idx_v, gbuf_a, gbuf_b, gbuf_c, sem_a, sem_b, sem_c,
                 mw_sh):
    cid = lax.axis_index("c")
    sid = lax.axis_index("s")
    wid = cid * NS + sid

    @pl.when(sid < NS - 1)
    def _():
        pltpu.sync_copy(mw_hbm.at[pl.ds(sid * RPS_A, RPS_A)],
                        mw_sh.at[pl.ds(sid * RPS_A, RPS_A)])

    @pl.when(sid == NS - 1)
    def _():
        pltpu.sync_copy(mw_hbm.at[pl.ds(15 * RPS_A, RPS_B)],
                        mw_sh.at[pl.ds(15 * RPS_A, RPS_B)])

    ebase = wid * EPW
    pltpu.sync_copy(src_hbm.at[pl.ds(ebase, EPW)], idx_v)
    plsc.subcore_barrier()

    def gath(c, buf, sem):
        pltpu.async_copy(mw_sh.at[idx_v.at[pl.ds(c * GS, GS)]], buf, sem)

    def wait_gath(c, buf, sem):
        pltpu.make_async_copy(mw_sh.at[idx_v.at[pl.ds(c * GS, GS)]],
                              buf, sem).wait()

    def store(c, buf):
        pltpu.sync_copy(buf, out_hbm.at[pl.ds(ebase + c * GS, GS)])

    gath(0, gbuf_a, sem_a)
    gath(1, gbuf_b, sem_b)

    def triple(i, carry):
        c = 3 * i
        gath(c + 2, gbuf_c, sem_c)
        wait_gath(c, gbuf_a, sem_a)
        store(c, gbuf_a)
        gath(c + 3, gbuf_a, sem_a)
        wait_gath(c + 1, gbuf_b, sem_b)
        store(c + 1, gbuf_b)
        gath(c + 4, gbuf_b, sem_b)
        wait_gath(c + 2, gbuf_c, sem_c)
        store(c + 2, gbuf_c)
        return carry

    lax.fori_loop(0, (NSCH - 2) // 3, triple, 0)

    c = NSCH - 2
    wait_gath(c, gbuf_a, sem_a)
    store(c, gbuf_a)
    wait_gath(c + 1, gbuf_b, sem_b)
    store(c + 1, gbuf_b)


_gather = pl.kernel(
    _gather_body,
    out_type=jax.ShapeDtypeStruct((NE, D), jnp.float32),
    mesh=_mesh,
    scratch_types=[
        pltpu.VMEM((EPW,), jnp.int32),
        pltpu.VMEM((GS, D), jnp.float32),
        pltpu.VMEM((GS, D), jnp.float32),
        pltpu.VMEM((GS, D), jnp.float32),
        pltpu.SemaphoreType.DMA,
        pltpu.SemaphoreType.DMA,
        pltpu.SemaphoreType.DMA,
        pltpu.VMEM_SHARED((NN, D), jnp.float32),
    ],
)


def _mailw_body(parts_ref, w_ref, o_ref):
    p = parts_ref[0] + parts_ref[1]
    o_ref[...] = jnp.dot(p, w_ref[...], preferred_element_type=jnp.float32)


def _fused_body(g_ref, eh_ref, ei_ref, w_ref, o_ref):
    o_ref[...] = (g_ref[...]
                  - jnp.dot(eh_ref[...], w_ref[...],
                            preferred_element_type=jnp.float32)
                  + ei_ref[...])


_MAILW_BLK = 1000
_FUSE_BLK = 6400


def kernel(edge_hidden, edge_init, W, edge_index):
    src = edge_index[0]
    dst = edge_index[1].reshape(NW, ISROWS, GS)
    zeros = jnp.zeros((NN, D), jnp.float32)

    parts = _scatter(edge_hidden, dst, zeros)

    mail_w = pl.pallas_call(
        _mailw_body,
        grid=(NN // _MAILW_BLK,),
        in_specs=[
            pl.BlockSpec((NC, _MAILW_BLK, D), lambda i: (0, i, 0)),
            pl.BlockSpec((D, D), lambda i: (0, 0)),
        ],
        out_specs=pl.BlockSpec((_MAILW_BLK, D), lambda i: (i, 0)),
        out_shape=jax.ShapeDtypeStruct((NN, D), jnp.float32),
    )(parts, W)

    gathered = _gather(mail_w, src)

    out = pl.pallas_call(
        _fused_body,
        grid=(NE // _FUSE_BLK,),
        in_specs=[
            pl.BlockSpec((_FUSE_BLK, D), lambda i: (i, 0)),
            pl.BlockSpec((_FUSE_BLK, D), lambda i: (i, 0)),
            pl.BlockSpec((_FUSE_BLK, D), lambda i: (i, 0)),
            pl.BlockSpec((D, D), lambda i: (0, 0)),
        ],
        out_specs=pl.BlockSpec((_FUSE_BLK, D), lambda i: (i, 0)),
        out_shape=jax.ShapeDtypeStruct((NE, D), jnp.float32),
    )(gathered, edge_hidden, edge_init, W)

    return out
